# initial kernel scaffold (unmeasured)
import jax
import jax.numpy as jnp
from jax import lax
from jax.experimental import pallas as pl
from jax.experimental.pallas import tpu as pltpu

B = 8
H = 8
D = 128
BS = 16
NB = 512
P_LOC = 512
T_LOC = P_LOC * BS


def kernel(Q, K, V, bt, lens):
    lens2 = lens.reshape(B, 1)

    def body(q_ref, k_ref, v_ref, bt_ref, lens_ref, out_ref,
             o_send, o_recv, ml_send, ml_recv,
             sem_o_send, sem_o_recv, sem_ml_send, sem_ml_recv):
        my_x = lax.axis_index("x")
        my_y = lax.axis_index("y")
        my_z = lax.axis_index("z")
        partner = (my_x, 1 - my_y, my_z)

        j_iota = lax.broadcasted_iota(jnp.int32, (B, NB), 1)
        valid = j_iota < lens_ref[...]
        bt_m = jnp.where(valid, bt_ref[...], -1)
        pid = my_y * P_LOC + lax.broadcasted_iota(jnp.int32, (B, NB, P_LOC), 2)
        w = (bt_m[:, :, None] == pid).astype(jnp.float32).sum(axis=1)
        wt = jnp.broadcast_to(w[:, :, None], (B, P_LOC, BS)).reshape(B, T_LOC)

        qs = q_ref[...][:, 0]
        k_all = k_ref[...].reshape(T_LOC, H, D)
        v_all = v_ref[...].reshape(T_LOC, H, D)

        s = jnp.einsum("bhd,khd->bhk", qs, k_all,
                       preferred_element_type=jnp.float32) * (D ** -0.5)
        m = s.max(axis=-1)
        e = jnp.exp(s - m[:, :, None]) * wt[:, None, :]
        l = e.sum(axis=-1)
        o = jnp.einsum("bhk,khd->bhd", e, v_all,
                       preferred_element_type=jnp.float32)

        o_send[...] = o
        ml_send[0] = m
        ml_send[1] = l

        barrier = pltpu.get_barrier_semaphore()
        pl.semaphore_signal(barrier, inc=1, device_id=partner,
                            device_id_type=pl.DeviceIdType.MESH)
        pl.semaphore_wait(barrier, 1)

        rdma_o = pltpu.make_async_remote_copy(
            src_ref=o_send, dst_ref=o_recv,
            send_sem=sem_o_send, recv_sem=sem_o_recv,
            device_id=partner, device_id_type=pl.DeviceIdType.MESH)
        rdma_ml = pltpu.make_async_remote_copy(
            src_ref=ml_send, dst_ref=ml_recv,
            send_sem=sem_ml_send, recv_sem=sem_ml_recv,
            device_id=partner, device_id_type=pl.DeviceIdType.MESH)
        rdma_o.start()
        rdma_ml.start()
        rdma_o.wait()
        rdma_ml.wait()

        m_o = ml_recv[0]
        l_o = ml_recv[1]
        m_g = jnp.maximum(m, m_o)
        c_s = jnp.exp(m - m_g)
        c_o = jnp.exp(m_o - m_g)
        l_g = l * c_s + l_o * c_o
        out = (o * c_s[:, :, None] + o_recv[...] * c_o[:, :, None]) / l_g[:, :, None]
        out_ref[...] = out.reshape(B, 1, H, D)

    return pl.pallas_call(
        body,
        out_shape=jax.ShapeDtypeStruct((B, 1, H, D), jnp.float32),
        in_specs=[pl.BlockSpec(memory_space=pltpu.VMEM)] * 5,
        out_specs=pl.BlockSpec(memory_space=pltpu.VMEM),
        scratch_shapes=[
            pltpu.VMEM((B, H, D), jnp.float32),
            pltpu.VMEM((B, H, D), jnp.float32),
            pltpu.VMEM((2, B, H), jnp.float32),
            pltpu.VMEM((2, B, H), jnp.float32),
            pltpu.SemaphoreType.DMA,
            pltpu.SemaphoreType.DMA,
            pltpu.SemaphoreType.DMA,
            pltpu.SemaphoreType.DMA,
        ],
        compiler_params=pltpu.CompilerParams(collective_id=0),
    )(Q, K, V, bt, lens2)


# baseline (device time: 31062 ns/iter reference)
import jax
import jax.numpy as jnp
from jax import lax
from jax.experimental import pallas as pl
from jax.experimental.pallas import tpu as pltpu

B = 8
H = 8
D = 128
BS = 16
NB = 512
P_LOC = 512
T_LOC = P_LOC * BS


def kernel(Q, K, V, bt, lens):
    lens2 = lens.reshape(B, 1)

    def body(q_ref, k_hbm, v_hbm, bt_ref, lens_ref, out_ref,
             kh_buf, vh_buf, o_send, o_recv, ml_send, ml_recv,
             copy_sems, sem_o_send, sem_o_recv, sem_ml_send, sem_ml_recv):
        my_x = lax.axis_index("x")
        my_y = lax.axis_index("y")
        my_z = lax.axis_index("z")
        partner = (my_x, 1 - my_y, my_z)

        def start_head_copy(h):
            slot = h % 2
            kc = pltpu.make_async_copy(
                k_hbm.at[:, :, h, :], kh_buf.at[slot], copy_sems.at[slot, 0])
            vc = pltpu.make_async_copy(
                v_hbm.at[:, :, h, :], vh_buf.at[slot], copy_sems.at[slot, 1])
            kc.start()
            vc.start()
            return kc, vc

        def wait_head_copy(h):
            slot = h % 2
            pltpu.make_async_copy(
                k_hbm.at[:, :, h, :], kh_buf.at[slot], copy_sems.at[slot, 0]
            ).wait()
            pltpu.make_async_copy(
                v_hbm.at[:, :, h, :], vh_buf.at[slot], copy_sems.at[slot, 1]
            ).wait()

        start_head_copy(0)

        j_iota = lax.broadcasted_iota(jnp.int32, (B, NB), 1)
        valid = j_iota < lens_ref[...]
        bt_m = jnp.where(valid, bt_ref[...], -1)
        pid = my_y * P_LOC + lax.broadcasted_iota(jnp.int32, (B, NB, P_LOC), 2)
        w = (bt_m[:, :, None] == pid).astype(jnp.float32).sum(axis=1)
        wt = jnp.broadcast_to(w[:, :, None], (B, P_LOC, BS)).reshape(B, T_LOC)

        qs = q_ref[...][:, 0]
        scale = D ** -0.5

        m_cols = []
        l_cols = []
        o_heads = []
        for h in range(H):
            if h + 1 < H:
                start_head_copy(h + 1)
            wait_head_copy(h)
            slot = h % 2
            kh = kh_buf[slot].reshape(T_LOC, D)
            vh = vh_buf[slot].reshape(T_LOC, D)
            q_h = qs[:, h, :]
            s = lax.dot_general(
                q_h, kh, (((1,), (1,)), ((), ())),
                preferred_element_type=jnp.float32) * scale
            m_h = s.max(axis=1, keepdims=True)
            e = jnp.exp(s - m_h) * wt
            l_h = e.sum(axis=1, keepdims=True)
            o_h = lax.dot_general(
                e, vh, (((1,), (0,)), ((), ())),
                preferred_element_type=jnp.float32)
            m_cols.append(m_h)
            l_cols.append(l_h)
            o_heads.append(o_h[:, None, :])

        m = jnp.concatenate(m_cols, axis=1)
        l = jnp.concatenate(l_cols, axis=1)
        o = jnp.concatenate(o_heads, axis=1)

        o_send[...] = o
        ml_send[0] = m
        ml_send[1] = l

        barrier = pltpu.get_barrier_semaphore()
        pl.semaphore_signal(barrier, inc=1, device_id=partner,
                            device_id_type=pl.DeviceIdType.MESH)
        pl.semaphore_wait(barrier, 1)

        rdma_o = pltpu.make_async_remote_copy(
            src_ref=o_send, dst_ref=o_recv,
            send_sem=sem_o_send, recv_sem=sem_o_recv,
            device_id=partner, device_id_type=pl.DeviceIdType.MESH)
        rdma_ml = pltpu.make_async_remote_copy(
            src_ref=ml_send, dst_ref=ml_recv,
            send_sem=sem_ml_send, recv_sem=sem_ml_recv,
            device_id=partner, device_id_type=pl.DeviceIdType.MESH)
        rdma_o.start()
        rdma_ml.start()
        rdma_o.wait()
        rdma_ml.wait()

        m_o = ml_recv[0]
        l_o = ml_recv[1]
        m_g = jnp.maximum(m, m_o)
        c_s = jnp.exp(m - m_g)
        c_o = jnp.exp(m_o - m_g)
        l_g = l * c_s + l_o * c_o
        out = (o * c_s[:, :, None] + o_recv[...] * c_o[:, :, None]) / l_g[:, :, None]
        out_ref[...] = out.reshape(B, 1, H, D)

    return pl.pallas_call(
        body,
        out_shape=jax.ShapeDtypeStruct((B, 1, H, D), jnp.float32),
        in_specs=[
            pl.BlockSpec(memory_space=pltpu.VMEM),
            pl.BlockSpec(memory_space=pltpu.MemorySpace.HBM),
            pl.BlockSpec(memory_space=pltpu.MemorySpace.HBM),
            pl.BlockSpec(memory_space=pltpu.VMEM),
            pl.BlockSpec(memory_space=pltpu.VMEM),
        ],
        out_specs=pl.BlockSpec(memory_space=pltpu.VMEM),
        scratch_shapes=[
            pltpu.VMEM((2, P_LOC, BS, D), jnp.float32),
            pltpu.VMEM((2, P_LOC, BS, D), jnp.float32),
            pltpu.VMEM((B, H, D), jnp.float32),
            pltpu.VMEM((B, H, D), jnp.float32),
            pltpu.VMEM((2, B, H), jnp.float32),
            pltpu.VMEM((2, B, H), jnp.float32),
            pltpu.SemaphoreType.DMA((2, 2)),
            pltpu.SemaphoreType.DMA,
            pltpu.SemaphoreType.DMA,
            pltpu.SemaphoreType.DMA,
            pltpu.SemaphoreType.DMA,
        ],
        compiler_params=pltpu.CompilerParams(collective_id=0),
    )(Q, K, V, bt, lens2)
